# Initial kernel scaffold; baseline (speedup 1.0000x reference)
import jax
import jax.numpy as jnp
from jax.experimental import pallas as pl


def kernel(x, pe_w, pe_b, q_w, q_b, k_w, k_b, v_w, v_b, proj_w, proj_b, gate_w, gate_b):
    raise NotImplementedError("write your pallas kernel here")


# speedup vs baseline: 1.5046x; 1.5046x over previous
import functools

import jax
import jax.numpy as jnp
from jax import lax
from jax.experimental import pallas as pl
from jax.experimental.pallas import tpu as pltpu
from jax.experimental.pallas import tpu_sc as plsc

_DIM = 96
_WS = 8
_GS = 9
_HID = 32
_GSW = _GS * _GS
_WSP = _WS * _WS
_NPIX = _GSW * _WSP
_CS = _WSP
_HALF = _CS // 2


def _shift2(x, di, dj):
    h, w = x.shape
    if di == -1:
        x = jnp.concatenate([jnp.zeros((1, w), x.dtype), x[:-1, :]], axis=0)
    elif di == 1:
        x = jnp.concatenate([x[1:, :], jnp.zeros((1, w), x.dtype)], axis=0)
    if dj == -1:
        x = jnp.concatenate([jnp.zeros((h, 1), x.dtype), x[:, :-1]], axis=1)
    elif dj == 1:
        x = jnp.concatenate([x[:, 1:], jnp.zeros((h, 1), x.dtype)], axis=1)
    return x


def _conv_body(x_ref, w_ref, b_ref, o_ref):
    x = x_ref[0, 0]
    acc = jnp.zeros_like(x)
    for a in range(3):
        for b in range(3):
            wk = w_ref[0, 0:1, a * 3 + b:a * 3 + b + 1]
            xs = _shift2(x, a - 1, b - 1).astype(jnp.bfloat16).astype(
                jnp.float32)
            acc = acc + wk * xs
    o_ref[0, 0] = (x + acc) + b_ref[0, 0:1, 0:1]


def _pe_conv(x, pe_w, pe_b):
    bsz, c, h, w = x.shape
    wt = pe_w.reshape(c, 1, 9)
    bt = pe_b.reshape(c, 1, 1)
    return pl.pallas_call(
        _conv_body,
        grid=(bsz, c),
        in_specs=[
            pl.BlockSpec((1, 1, h, w), lambda b_, c_: (b_, c_, 0, 0)),
            pl.BlockSpec((1, 1, 9), lambda b_, c_: (c_, 0, 0)),
            pl.BlockSpec((1, 1, 1), lambda b_, c_: (c_, 0, 0)),
        ],
        out_specs=pl.BlockSpec((1, 1, h, w), lambda b_, c_: (b_, c_, 0, 0)),
        out_shape=jax.ShapeDtypeStruct((bsz, c, h, w), x.dtype),
    )(x, wt, bt)


def _prep_body(xg_ref, pos_ref, cum_ref, oh_ref, prior_ref):
    g = pl.program_id(0)
    xg = xg_ref[0]
    xg3 = xg.reshape(_GSW, _WSP, _DIM)
    means = jnp.mean(xg3, axis=1)
    sim = lax.dot_general(xg, means, (((1,), (1,)), ((), ())),
                          preferred_element_type=jnp.float32)
    lane = lax.broadcasted_iota(jnp.int32, (_NPIX, _GSW), 1)
    mx = jnp.max(sim, axis=1, keepdims=True)
    assign = jnp.min(jnp.where(sim == mx, lane, _GSW), axis=1,
                     keepdims=True)
    onehot = (lane == assign).astype(jnp.float32)

    o3 = onehot.reshape(_GSW, _CS, _GSW)
    hc = jnp.sum(o3, axis=1)
    h = jnp.sum(hc, axis=0, keepdims=True)
    r81 = lax.broadcasted_iota(jnp.int32, (_GSW, _GSW), 0)
    c81 = lax.broadcasted_iota(jnp.int32, (_GSW, _GSW), 1)
    lstrict = (c81 < r81).astype(jnp.float32)
    cum_incl = h
    s = 1
    while s < _GSW:
        cum_incl = cum_incl + jnp.concatenate(
            [jnp.zeros((1, s), jnp.float32), cum_incl[:, :_GSW - s]], axis=1)
        s *= 2
    off = cum_incl - h
    prior = lax.dot_general(lstrict, hc, (((1,), (0,)), ((), ())),
                            preferred_element_type=jnp.float32)
    oh_ref[...] = onehot
    prior_ref[...] = prior + off

    r64 = lax.broadcasted_iota(jnp.int32, (_CS, _CS), 0)
    c64 = lax.broadcasted_iota(jnp.int32, (_CS, _CS), 1)
    l64 = (c64 < r64).astype(jnp.float32)

    base = (g * _NPIX).astype(jnp.float32)

    def chunk(c, _):
        oc = oh_ref[pl.ds(c * _CS, _CS), :]
        rank = lax.dot_general(l64, oc, (((1,), (0,)), ((), ())),
                               preferred_element_type=jnp.float32)
        tot = rank + prior_ref[pl.ds(c, 1), :]
        posc = jnp.sum(oc * tot, axis=1, keepdims=True) + base
        pos_ref[0, pl.ds(c * _CS, _CS), :] = posc.astype(jnp.int32)
        return 0

    lax.fori_loop(0, _GSW, chunk, 0)
    cum_ref[0] = cum_incl.astype(jnp.int32)


def _prep(xg):
    ng = xg.shape[0]
    return pl.pallas_call(
        _prep_body,
        grid=(ng,),
        in_specs=[pl.BlockSpec((1, _NPIX, _DIM), lambda g: (g, 0, 0))],
        out_specs=[
            pl.BlockSpec((1, _NPIX, 1), lambda g: (g, 0, 0)),
            pl.BlockSpec((1, 1, _GSW), lambda g: (g, 0, 0)),
        ],
        out_shape=[
            jax.ShapeDtypeStruct((ng, _NPIX, 1), jnp.int32),
            jax.ShapeDtypeStruct((ng, 1, _GSW), jnp.int32),
        ],
        scratch_shapes=[
            pltpu.VMEM((_NPIX, _GSW), jnp.float32),
            pltpu.VMEM((_GSW, _GSW), jnp.float32),
        ],
    )(xg)


def _attn_body(xs_ref, xf_ref, cum_ref, qw_ref, qb_ref, kw_ref, kb_ref,
               vw_ref, vb_ref, gw_ref, gb_ref, pw_ref, pb_ref, o_ref,
               z_ref, q_ref, kp_ref, vp_ref, qid_ref, kvid_ref):
    xs = xs_ref[0]
    xf = xf_ref[0]
    cum = cum_ref[0]

    def proj(xmat, w_ref, b_ref):
        return lax.dot_general(
            xmat, w_ref[...], (((1,), (1,)), ((), ())),
            preferred_element_type=jnp.float32) + b_ref[...]

    q_ref[...] = proj(xs, qw_ref, qb_ref)
    k = proj(xs, kw_ref, kb_ref)
    v = proj(xs, vw_ref, vb_ref)
    zpadk = jnp.zeros((_HALF, _HID), jnp.float32)
    zpadv = jnp.zeros((_HALF, _DIM), jnp.float32)
    kp_ref[...] = jnp.concatenate([zpadk, k, zpadk], axis=0)
    vp_ref[...] = jnp.concatenate([zpadv, v, zpadv], axis=0)

    jcol = lax.broadcasted_iota(jnp.int32, (_NPIX, _GSW), 0)
    qid_ref[...] = jnp.sum((cum <= jcol).astype(jnp.int32), axis=1,
                           keepdims=True)
    crow = lax.broadcasted_iota(jnp.int32, (_GSW, 2 * _CS), 0)
    trow = lax.broadcasted_iota(jnp.int32, (_GSW, 2 * _CS), 1)
    jkv = crow * _CS + trow - _HALF
    kvvalid = (jkv >= 0) & (jkv < _NPIX)
    kvid3 = jnp.sum(
        (cum.reshape(1, 1, _GSW) <= jkv[:, :, None]).astype(jnp.int32),
        axis=2)
    kvid_ref[...] = jnp.where(kvvalid, kvid3, -1)

    scale = float(_DIM) ** (-0.5)

    def chunk(c, _):
        qc = q_ref[pl.ds(c * _CS, _CS), :]
        kc = kp_ref[pl.ds(c * _CS, 2 * _CS), :]
        vc = vp_ref[pl.ds(c * _CS, 2 * _CS), :]
        att = lax.dot_general(qc, kc, (((1,), (1,)), ((), ())),
                              preferred_element_type=jnp.float32) * scale
        m = qid_ref[pl.ds(c * _CS, _CS), :] == kvid_ref[pl.ds(c, 1), :]
        att = jnp.where(m, att, -10000.0)
        att = att - jnp.max(att, axis=1, keepdims=True)
        e = jnp.exp(att)
        att = e / jnp.sum(e, axis=1, keepdims=True)
        z_ref[pl.ds(c * _CS, _CS), :] = lax.dot_general(
            att, vc, (((1,), (0,)), ((), ())),
            preferred_element_type=jnp.float32)
        return 0

    lax.fori_loop(0, _GSW, chunk, 0)

    glin = proj(xf, gw_ref, gb_ref)
    gate = 1.0 / (1.0 + jnp.exp(-glin))
    zg = z_ref[...] * gate
    o_ref[0] = lax.dot_general(zg, pw_ref[...], (((1,), (1,)), ((), ())),
                               preferred_element_type=jnp.float32) + pb_ref[...]


def _attn(xs, xg, cum, q_w, q_b, k_w, k_b, v_w, v_b, gate_w, gate_b,
          proj_w, proj_b):
    ng = xs.shape[0]
    wspec = lambda shape: pl.BlockSpec(shape, lambda g: (0,) * len(shape))
    return pl.pallas_call(
        _attn_body,
        grid=(ng,),
        in_specs=[
            pl.BlockSpec((1, _NPIX, _DIM), lambda g: (g, 0, 0)),
            pl.BlockSpec((1, _NPIX, _DIM), lambda g: (g, 0, 0)),
            pl.BlockSpec((1, 1, _GSW), lambda g: (g, 0, 0)),
            wspec((_HID, _DIM)), wspec((1, _HID)),
            wspec((_HID, _DIM)), wspec((1, _HID)),
            wspec((_DIM, _DIM)), wspec((1, _DIM)),
            wspec((_DIM, _DIM)), wspec((1, _DIM)),
            wspec((_DIM, _DIM)), wspec((1, _DIM)),
        ],
        out_specs=pl.BlockSpec((1, _NPIX, _DIM), lambda g: (g, 0, 0)),
        out_shape=jax.ShapeDtypeStruct((ng, _NPIX, _DIM), jnp.float32),
        scratch_shapes=[
            pltpu.VMEM((_NPIX, _DIM), jnp.float32),
            pltpu.VMEM((_NPIX, _HID), jnp.float32),
            pltpu.VMEM((_NPIX + _CS, _HID), jnp.float32),
            pltpu.VMEM((_NPIX + _CS, _DIM), jnp.float32),
            pltpu.VMEM((_NPIX, 1), jnp.int32),
            pltpu.VMEM((_GSW, 2 * _CS), jnp.int32),
        ],
    )(xs, xg, cum, q_w, q_b.reshape(1, -1), k_w, k_b.reshape(1, -1),
      v_w, v_b.reshape(1, -1), gate_w, gate_b.reshape(1, -1),
      proj_w, proj_b.reshape(1, -1))


_SC_NC = 2
_SC_NS = 16
_SC_NW = _SC_NC * _SC_NS
_SC_CH = 128


def _sc_permute(src, idx, gather):
    n, d = src.shape
    nch = n // _SC_CH
    trips = (nch + _SC_NW - 1) // _SC_NW
    mesh = plsc.VectorSubcoreMesh(core_axis_name="c", subcore_axis_name="s")

    @functools.partial(
        pl.kernel, mesh=mesh,
        compiler_params=pltpu.CompilerParams(use_tc_tiling_on_sc=False),
        out_type=jax.ShapeDtypeStruct((n, d), jnp.float32),
        scratch_types=[
            pltpu.VMEM((_SC_CH,), jnp.int32),
            pltpu.VMEM((_SC_CH, d), jnp.float32),
            pltpu.SemaphoreType.DMA,
        ],
    )
    def run(src_hbm, idx_hbm, out_hbm, idx_v, rows_v, sem):
        w = lax.axis_index("s") * _SC_NC + lax.axis_index("c")

        def body(i, _):
            c = w + _SC_NW * i

            @pl.when(c < nch)
            def _():
                base = c * _SC_CH
                pltpu.sync_copy(idx_hbm.at[pl.ds(base, _SC_CH)], idx_v)
                if gather:
                    pltpu.async_copy(src_hbm.at[idx_v], rows_v, sem).wait()
                    pltpu.sync_copy(rows_v, out_hbm.at[pl.ds(base, _SC_CH)])
                else:
                    pltpu.sync_copy(src_hbm.at[pl.ds(base, _SC_CH)], rows_v)
                    pltpu.async_copy(rows_v, out_hbm.at[idx_v], sem).wait()
            return 0

        lax.fori_loop(0, trips, body, 0)

    return run(src, idx)


def kernel(x, pe_w, pe_b, q_w, q_b, k_w, k_b, v_w, v_b, proj_w, proj_b,
           gate_w, gate_b):
    bsz, c, h, w = x.shape
    hw, ww = h // _WS, w // _WS
    gh, gw = hw // _GS, ww // _GS
    ng = bsz * gh * gw

    y = _pe_conv(x, pe_w, pe_b)

    xg = y.reshape(bsz, c, hw, _WS, ww, _WS)
    xg = jnp.transpose(xg, (0, 2, 4, 3, 5, 1)).reshape(bsz, hw, ww, _WSP, c)
    xg = xg.reshape(bsz, gh, _GS, gw, _GS, _WSP, c)
    xg = jnp.transpose(xg, (0, 1, 3, 2, 4, 5, 6)).reshape(ng, _NPIX, c)

    pos, cum = _prep(xg)
    posf = pos.reshape(ng * _NPIX)
    xflat = xg.reshape(ng * _NPIX, c)

    xs = _sc_permute(xflat, posf, gather=False)
    zp = _attn(xs.reshape(ng, _NPIX, c), xg, cum, q_w, q_b, k_w, k_b,
               v_w, v_b, gate_w, gate_b, proj_w, proj_b)
    fin = _sc_permute(zp.reshape(ng * _NPIX, c), posf, gather=True)

    out = fin.reshape(bsz, gh, gw, _GS, _GS, _WS, _WS, c)
    out = jnp.transpose(out, (0, 7, 1, 3, 5, 2, 4, 6)).reshape(bsz, c, h, w)
    return out


# speedup vs baseline: 2.3031x; 1.5307x over previous
import functools

import jax
import jax.numpy as jnp
from jax import lax
from jax.experimental import pallas as pl
from jax.experimental.pallas import tpu as pltpu
from jax.experimental.pallas import tpu_sc as plsc

_DIM = 96
_WS = 8
_GS = 9
_HID = 32
_GSW = _GS * _GS
_WSP = _WS * _WS
_NPIX = _GSW * _WSP
_CS = _WSP
_HALF = _CS // 2


def _shift2(x, di, dj):
    h, w = x.shape
    if di == -1:
        x = jnp.concatenate([jnp.zeros((1, w), x.dtype), x[:-1, :]], axis=0)
    elif di == 1:
        x = jnp.concatenate([x[1:, :], jnp.zeros((1, w), x.dtype)], axis=0)
    if dj == -1:
        x = jnp.concatenate([jnp.zeros((h, 1), x.dtype), x[:, :-1]], axis=1)
    elif dj == 1:
        x = jnp.concatenate([x[:, 1:], jnp.zeros((h, 1), x.dtype)], axis=1)
    return x


def _conv_body(x_ref, w_ref, b_ref, o_ref):
    x = x_ref[0, 0]
    acc = jnp.zeros_like(x)
    for a in range(3):
        for b in range(3):
            wk = w_ref[0, 0:1, a * 3 + b:a * 3 + b + 1]
            xs = _shift2(x, a - 1, b - 1).astype(jnp.bfloat16).astype(
                jnp.float32)
            acc = acc + wk * xs
    o_ref[0, 0] = (x + acc) + b_ref[0, 0:1, 0:1]


def _pe_conv(x, pe_w, pe_b):
    bsz, c, h, w = x.shape
    wt = pe_w.reshape(c, 1, 9)
    bt = pe_b.reshape(c, 1, 1)
    return pl.pallas_call(
        _conv_body,
        grid=(bsz, c),
        in_specs=[
            pl.BlockSpec((1, 1, h, w), lambda b_, c_: (b_, c_, 0, 0)),
            pl.BlockSpec((1, 1, 9), lambda b_, c_: (c_, 0, 0)),
            pl.BlockSpec((1, 1, 1), lambda b_, c_: (c_, 0, 0)),
        ],
        out_specs=pl.BlockSpec((1, 1, h, w), lambda b_, c_: (b_, c_, 0, 0)),
        out_shape=jax.ShapeDtypeStruct((bsz, c, h, w), x.dtype),
    )(x, wt, bt)


def _prep_body(xg_ref, pos_ref, cum_ref, oh_ref, prior_ref):
    g = pl.program_id(0)
    xg = xg_ref[0]
    xg3 = xg.reshape(_GSW, _WSP, _DIM)
    means = jnp.mean(xg3, axis=1)
    sim = lax.dot_general(xg, means, (((1,), (1,)), ((), ())),
                          preferred_element_type=jnp.float32)
    lane = lax.broadcasted_iota(jnp.int32, (_NPIX, _GSW), 1)
    mx = jnp.max(sim, axis=1, keepdims=True)
    assign = jnp.min(jnp.where(sim == mx, lane, _GSW), axis=1,
                     keepdims=True)
    onehot = (lane == assign).astype(jnp.float32)

    o3 = onehot.reshape(_GSW, _CS, _GSW)
    hc = jnp.sum(o3, axis=1)
    h = jnp.sum(hc, axis=0, keepdims=True)
    r81 = lax.broadcasted_iota(jnp.int32, (_GSW, _GSW), 0)
    c81 = lax.broadcasted_iota(jnp.int32, (_GSW, _GSW), 1)
    lstrict = (c81 < r81).astype(jnp.float32)
    cum_incl = h
    s = 1
    while s < _GSW:
        cum_incl = cum_incl + jnp.concatenate(
            [jnp.zeros((1, s), jnp.float32), cum_incl[:, :_GSW - s]], axis=1)
        s *= 2
    off = cum_incl - h
    prior = lax.dot_general(lstrict, hc, (((1,), (0,)), ((), ())),
                            preferred_element_type=jnp.float32)
    oh_ref[...] = onehot
    prior_exp = jnp.broadcast_to(
        prior[:, None, :], (_GSW, _CS, _GSW)).reshape(_NPIX, _GSW)
    base = (g * _NPIX).astype(jnp.float32)
    pick = jnp.sum(onehot * (prior_exp + off), axis=1, keepdims=True) + base
    prior_ref[...] = pick

    blk = 9 * _CS
    rb = lax.broadcasted_iota(jnp.int32, (blk, blk), 0)
    cb = lax.broadcasted_iota(jnp.int32, (blk, blk), 1)
    lblk = ((cb < rb) & (cb // _CS == rb // _CS)).astype(jnp.float32)

    def chunk(c, _):
        oc = oh_ref[pl.ds(c * blk, blk), :]
        rank = lax.dot_general(lblk, oc, (((1,), (0,)), ((), ())),
                               preferred_element_type=jnp.float32)
        posc = (jnp.sum(oc * rank, axis=1, keepdims=True)
                + prior_ref[pl.ds(c * blk, blk), :])
        pos_ref[0, pl.ds(c * blk, blk), :] = posc.astype(jnp.int32)
        return 0

    lax.fori_loop(0, _NPIX // blk, chunk, 0)
    cum_ref[0] = cum_incl.astype(jnp.int32)


def _prep(xg):
    ng = xg.shape[0]
    return pl.pallas_call(
        _prep_body,
        grid=(ng,),
        in_specs=[pl.BlockSpec((1, _NPIX, _DIM), lambda g: (g, 0, 0))],
        out_specs=[
            pl.BlockSpec((1, _NPIX, 1), lambda g: (g, 0, 0)),
            pl.BlockSpec((1, 1, _GSW), lambda g: (g, 0, 0)),
        ],
        out_shape=[
            jax.ShapeDtypeStruct((ng, _NPIX, 1), jnp.int32),
            jax.ShapeDtypeStruct((ng, 1, _GSW), jnp.int32),
        ],
        scratch_shapes=[
            pltpu.VMEM((_NPIX, _GSW), jnp.float32),
            pltpu.VMEM((_NPIX, 1), jnp.float32),
        ],
    )(xg)


def _attn_body(xs_ref, xf_ref, cum_ref, qw_ref, qb_ref, kw_ref, kb_ref,
               vw_ref, vb_ref, gw_ref, gb_ref, pw_ref, pb_ref, o_ref,
               z_ref, q_ref, kp_ref, vp_ref, qid_ref, kvid_ref):
    xs = xs_ref[0]
    xf = xf_ref[0]
    cum = cum_ref[0]

    def proj(xmat, w_ref, b_ref):
        return lax.dot_general(
            xmat, w_ref[...], (((1,), (1,)), ((), ())),
            preferred_element_type=jnp.float32) + b_ref[...]

    q_ref[...] = proj(xs, qw_ref, qb_ref)
    k = proj(xs, kw_ref, kb_ref)
    v = proj(xs, vw_ref, vb_ref)
    kp_ref[...] = jnp.concatenate(
        [jnp.zeros((_HALF, _HID), jnp.float32), k,
         jnp.zeros((2 * _CS, _HID), jnp.float32)], axis=0)
    vp_ref[...] = jnp.concatenate(
        [jnp.zeros((_HALF, _DIM), jnp.float32), v,
         jnp.zeros((2 * _CS, _DIM), jnp.float32)], axis=0)

    jcol = lax.broadcasted_iota(jnp.int32, (_NPIX, _GSW), 0)
    qid_ref[...] = jnp.sum((cum <= jcol).astype(jnp.int32), axis=1,
                           keepdims=True)
    blk = 9 * _CS
    kvw = blk + 2 * _CS
    nblk = _NPIX // blk
    grow = lax.broadcasted_iota(jnp.int32, (nblk, kvw), 0)
    trow = lax.broadcasted_iota(jnp.int32, (nblk, kvw), 1)
    jkv = grow * blk + trow - _HALF
    kvvalid = (jkv >= 0) & (jkv < _NPIX)
    kvid3 = jnp.sum(
        (cum.reshape(1, 1, _GSW) <= jkv[:, :, None]).astype(jnp.int32),
        axis=2)
    kvid_ref[...] = jnp.where(kvvalid, kvid3, -1)

    arow = lax.broadcasted_iota(jnp.int32, (blk, kvw), 0)
    tcol = lax.broadcasted_iota(jnp.int32, (blk, kvw), 1)
    awin = (arow // _CS) * _CS
    wmask = (tcol >= awin) & (tcol < awin + 2 * _CS)

    scale = float(_DIM) ** (-0.5)

    def chunk(c, _):
        qc = q_ref[pl.ds(c * blk, blk), :]
        kc = kp_ref[pl.ds(c * blk, kvw), :]
        vc = vp_ref[pl.ds(c * blk, kvw), :]
        att = lax.dot_general(qc, kc, (((1,), (1,)), ((), ())),
                              preferred_element_type=jnp.float32) * scale
        m = wmask & (qid_ref[pl.ds(c * blk, blk), :]
                     == kvid_ref[pl.ds(c, 1), :])
        att = jnp.where(m, att, -10000.0)
        att = att - jnp.max(att, axis=1, keepdims=True)
        e = jnp.exp(att)
        att = e / jnp.sum(e, axis=1, keepdims=True)
        z_ref[pl.ds(c * blk, blk), :] = lax.dot_general(
            att, vc, (((1,), (0,)), ((), ())),
            preferred_element_type=jnp.float32)
        return 0

    lax.fori_loop(0, nblk, chunk, 0)

    glin = proj(xf, gw_ref, gb_ref)
    gate = 1.0 / (1.0 + jnp.exp(-glin))
    zg = z_ref[...] * gate
    o_ref[0] = lax.dot_general(zg, pw_ref[...], (((1,), (1,)), ((), ())),
                               preferred_element_type=jnp.float32) + pb_ref[...]


def _attn(xs, xg, cum, q_w, q_b, k_w, k_b, v_w, v_b, gate_w, gate_b,
          proj_w, proj_b):
    ng = xs.shape[0]
    wspec = lambda shape: pl.BlockSpec(shape, lambda g: (0,) * len(shape))
    return pl.pallas_call(
        _attn_body,
        grid=(ng,),
        in_specs=[
            pl.BlockSpec((1, _NPIX, _DIM), lambda g: (g, 0, 0)),
            pl.BlockSpec((1, _NPIX, _DIM), lambda g: (g, 0, 0)),
            pl.BlockSpec((1, 1, _GSW), lambda g: (g, 0, 0)),
            wspec((_HID, _DIM)), wspec((1, _HID)),
            wspec((_HID, _DIM)), wspec((1, _HID)),
            wspec((_DIM, _DIM)), wspec((1, _DIM)),
            wspec((_DIM, _DIM)), wspec((1, _DIM)),
            wspec((_DIM, _DIM)), wspec((1, _DIM)),
        ],
        out_specs=pl.BlockSpec((1, _NPIX, _DIM), lambda g: (g, 0, 0)),
        out_shape=jax.ShapeDtypeStruct((ng, _NPIX, _DIM), jnp.float32),
        scratch_shapes=[
            pltpu.VMEM((_NPIX, _DIM), jnp.float32),
            pltpu.VMEM((_NPIX, _HID), jnp.float32),
            pltpu.VMEM((_NPIX + _HALF + 2 * _CS, _HID), jnp.float32),
            pltpu.VMEM((_NPIX + _HALF + 2 * _CS, _DIM), jnp.float32),
            pltpu.VMEM((_NPIX, 1), jnp.int32),
            pltpu.VMEM((9, 9 * _CS + 2 * _CS), jnp.int32),
        ],
    )(xs, xg, cum, q_w, q_b.reshape(1, -1), k_w, k_b.reshape(1, -1),
      v_w, v_b.reshape(1, -1), gate_w, gate_b.reshape(1, -1),
      proj_w, proj_b.reshape(1, -1))


_SC_NC = 2
_SC_NS = 16
_SC_NW = _SC_NC * _SC_NS
_SC_CH = 128


def _sc_permute(src, idx, gather):
    n, d = src.shape
    nch = n // _SC_CH
    trips = (nch + _SC_NW - 1) // _SC_NW
    mesh = plsc.VectorSubcoreMesh(core_axis_name="c", subcore_axis_name="s")

    @functools.partial(
        pl.kernel, mesh=mesh,
        compiler_params=pltpu.CompilerParams(use_tc_tiling_on_sc=False),
        out_type=jax.ShapeDtypeStruct((n, d), jnp.float32),
        scratch_types=[
            pltpu.VMEM((_SC_CH,), jnp.int32),
            pltpu.VMEM((_SC_CH, d), jnp.float32),
            pltpu.SemaphoreType.DMA,
        ],
    )
    def run(src_hbm, idx_hbm, out_hbm, idx_v, rows_v, sem):
        w = lax.axis_index("s") * _SC_NC + lax.axis_index("c")

        def body(i, _):
            c = w + _SC_NW * i

            @pl.when(c < nch)
            def _():
                base = c * _SC_CH
                pltpu.sync_copy(idx_hbm.at[pl.ds(base, _SC_CH)], idx_v)
                if gather:
                    pltpu.async_copy(src_hbm.at[idx_v], rows_v, sem).wait()
                    pltpu.sync_copy(rows_v, out_hbm.at[pl.ds(base, _SC_CH)])
                else:
                    pltpu.sync_copy(src_hbm.at[pl.ds(base, _SC_CH)], rows_v)
                    pltpu.async_copy(rows_v, out_hbm.at[idx_v], sem).wait()
            return 0

        lax.fori_loop(0, trips, body, 0)

    return run(src, idx)


def kernel(x, pe_w, pe_b, q_w, q_b, k_w, k_b, v_w, v_b, proj_w, proj_b,
           gate_w, gate_b):
    bsz, c, h, w = x.shape
    hw, ww = h // _WS, w // _WS
    gh, gw = hw // _GS, ww // _GS
    ng = bsz * gh * gw

    y = _pe_conv(x, pe_w, pe_b)

    xg = y.reshape(bsz, c, hw, _WS, ww, _WS)
    xg = jnp.transpose(xg, (0, 2, 4, 3, 5, 1)).reshape(bsz, hw, ww, _WSP, c)
    xg = xg.reshape(bsz, gh, _GS, gw, _GS, _WSP, c)
    xg = jnp.transpose(xg, (0, 1, 3, 2, 4, 5, 6)).reshape(ng, _NPIX, c)

    pos, cum = _prep(xg)
    posf = pos.reshape(ng * _NPIX)
    xflat = xg.reshape(ng * _NPIX, c)

    xs = _sc_permute(xflat, posf, gather=False)
    zp = _attn(xs.reshape(ng, _NPIX, c), xg, cum, q_w, q_b, k_w, k_b,
               v_w, v_b, gate_w, gate_b, proj_w, proj_b)
    fin = _sc_permute(zp.reshape(ng * _NPIX, c), posf, gather=True)

    out = fin.reshape(bsz, gh, gw, _GS, _GS, _WS, _WS, c)
    out = jnp.transpose(out, (0, 7, 1, 3, 5, 2, 4, 6)).reshape(bsz, c, h, w)
    return out
